# Initial kernel scaffold; baseline (speedup 1.0000x reference)
import jax
import jax.numpy as jnp
from jax.experimental import pallas as pl


def kernel(x, edge_attr, edge_weight, Wq, Wk, Wv, We, Wskip, Wout, bout, edge_index, batch):
    raise NotImplementedError("write your pallas kernel here")


# speedup vs baseline: 11.2681x; 11.2681x over previous
import functools

import jax
import jax.numpy as jnp
import numpy as np
from jax import lax
from jax.experimental import pallas as pl
from jax.experimental.pallas import tpu as pltpu
from jax.experimental.pallas import tpu_sc as plsc

H = 8
NCORE = 2
NSUB = 16
CHUNK = 128


def _proj_body(x_ref, wq_ref, wk_ref, wv_ref, wskip_ref, wet_ref,
               q_ref, k_ref, v_ref, skip_ref, qwe_ref, *, scale, ch, edim,
               half):
    x = x_ref[...]
    q = jnp.dot(x, wq_ref[...], preferred_element_type=jnp.float32, precision=lax.Precision.HIGHEST) * scale
    k = jnp.dot(x, wk_ref[...], preferred_element_type=jnp.float32, precision=lax.Precision.HIGHEST)
    v = jnp.dot(x, wv_ref[...], preferred_element_type=jnp.float32, precision=lax.Precision.HIGHEST)
    s = jnp.dot(x, wskip_ref[...], preferred_element_type=jnp.float32, precision=lax.Precision.HIGHEST)
    half = (H // NCORE) * ch
    for c in range(NCORE):
        q_ref[c, :, :] = q[:, c * half:(c + 1) * half]
        k_ref[c, :, :] = k[:, c * half:(c + 1) * half]
        v_ref[c, :, :] = v[:, c * half:(c + 1) * half]
        skip_ref[c, :, :] = s[:, c * half:(c + 1) * half]
    wet = wet_ref[...]
    hh = H // NCORE
    nrows = x.shape[0]
    for c in range(NCORE):
        parts = []
        for hl in range(hh):
            h = c * hh + hl
            parts.append(jnp.dot(q[:, h * ch:(h + 1) * ch],
                                 wet[h * ch:(h + 1) * ch, :],
                                 preferred_element_type=jnp.float32, precision=lax.Precision.HIGHEST))
        parts.append(jnp.zeros((nrows, half - hh * edim), jnp.float32))
        qwe_ref[c, :, :] = jnp.concatenate(parts, axis=1)


def _gather_body(q2, k2, v2, qwe2, dst2, src2, qd, ks, vs, qwed,
                 idxd, idxs, bq, bk, bv, bw, sq, sk, sv, sw,
                 *, n_chunks, per_worker, e_pad):
    c = lax.axis_index("c")
    s = lax.axis_index("s")

    def body(i, carry):
        base = s * per_worker + i * CHUNK
        ebase = c * e_pad + base
        pltpu.sync_copy(dst2.at[pl.ds(ebase, CHUNK)], idxd)
        pltpu.sync_copy(src2.at[pl.ds(ebase, CHUNK)], idxs)
        h1 = pltpu.async_copy(q2.at[idxd], bq, sq)
        h2 = pltpu.async_copy(k2.at[idxs], bk, sk)
        h3 = pltpu.async_copy(v2.at[idxs], bv, sv)
        h4 = pltpu.async_copy(qwe2.at[idxd], bw, sw)
        h1.wait(); h2.wait(); h3.wait(); h4.wait()
        pltpu.sync_copy(bq, qd.at[c, pl.ds(base, CHUNK)])
        pltpu.sync_copy(bk, ks.at[c, pl.ds(base, CHUNK)])
        pltpu.sync_copy(bv, vs.at[c, pl.ds(base, CHUNK)])
        pltpu.sync_copy(bw, qwed.at[c, pl.ds(base, CHUNK)])
        return carry

    lax.fori_loop(0, n_chunks, body, 0)


def _alpha_body(qd_ref, ks_ref, qwed_ref, ea_ref, alpha_ref, pm_ref,
                *, be, n_edges, ch, edim):
    hh = H // NCORE
    s32 = (lax.broadcasted_iota(jnp.int32, (hh * ch, hh), 0) // ch ==
           lax.broadcasted_iota(jnp.int32, (hh * ch, hh), 1)).astype(jnp.float32)
    s12 = (lax.broadcasted_iota(jnp.int32, (hh * edim, hh), 0) // edim ==
           lax.broadcasted_iota(jnp.int32, (hh * edim, hh), 1)).astype(jnp.float32)
    eat = jnp.concatenate([ea_ref[...]] * hh, axis=1)
    pid = pl.program_id(0)
    valid = (pid * be + lax.broadcasted_iota(jnp.int32, (be, hh), 0)) < n_edges
    for c in range(NCORE):
        prod = qd_ref[c] * ks_ref[c]
        a = (jnp.dot(prod, s32, preferred_element_type=jnp.float32, precision=lax.Precision.HIGHEST) +
             jnp.dot(qwed_ref[c][:, :hh * edim] * eat, s12,
                     preferred_element_type=jnp.float32, precision=lax.Precision.HIGHEST))
        alpha_ref[c, :, :] = a
        pm_ref[0, c, :] = jnp.max(jnp.where(valid, a, -1e30), axis=0)


def _payload_body(alpha_ref, pm_ref, vs_ref, ea_ref, ew_ref,
                  msgv_ref, pay2_ref, *, be, n_edges, ch, edim):
    hh = H // NCORE
    m = jnp.max(pm_ref[...], axis=0)
    pid = pl.program_id(0)
    valid = (pid * be + lax.broadcasted_iota(jnp.int32, (be, hh), 0)) < n_edges
    r32 = (lax.broadcasted_iota(jnp.int32, (hh, hh * ch), 1) // ch ==
           lax.broadcasted_iota(jnp.int32, (hh, hh * ch), 0)).astype(jnp.float32)
    r12 = (lax.broadcasted_iota(jnp.int32, (hh, hh * edim), 1) // edim ==
           lax.broadcasted_iota(jnp.int32, (hh, hh * edim), 0)).astype(jnp.float32)
    eat = jnp.concatenate([ea_ref[...]] * hh, axis=1)
    ew = ew_ref[...]
    pad = hh * ch - hh * edim - hh
    for c in range(NCORE):
        eaz = jnp.where(valid, jnp.exp(alpha_ref[c] - m[c]), 0.0)
        eaw = eaz * ew
        msgv_ref[c, :, :] = vs_ref[c] * jnp.dot(eaw, r32,
                                                preferred_element_type=jnp.float32, precision=lax.Precision.HIGHEST)
        p48 = eat * jnp.dot(eaw, r12, preferred_element_type=jnp.float32, precision=lax.Precision.HIGHEST)
        pay2_ref[c, :, :] = jnp.concatenate(
            [p48, eaz, jnp.zeros((be, pad), jnp.float32)], axis=1)


def _scatter_body(pay, dstp, zrows, outp, acc, idx, bv,
                  *, n_chunks, per_worker, rows_per_tile):
    c = lax.axis_index("c")
    s = lax.axis_index("s")
    rows0 = s * rows_per_tile
    pltpu.sync_copy(zrows.at[pl.ds(0, rows_per_tile)],
                    acc.at[pl.ds(rows0, rows_per_tile)])
    plsc.subcore_barrier()

    def body(i, carry):
        base = s * per_worker + i * CHUNK
        pltpu.sync_copy(dstp.at[pl.ds(base, CHUNK)], idx)
        pltpu.sync_copy(pay.at[c, pl.ds(base, CHUNK)], bv)
        pltpu.sync_copy(bv, acc.at[idx], add=True)
        return carry

    lax.fori_loop(0, n_chunks, body, 0)
    plsc.subcore_barrier()
    pltpu.sync_copy(acc.at[pl.ds(rows0, rows_per_tile)],
                    outp.at[c, pl.ds(rows0, rows_per_tile)])


def _combine_body(outv_ref, t2_ref, skip_ref, wes_ref, batch_ref,
                  sums_ref, counts_ref, *, bn, nb, ch, edim):
    hh = H // NCORE
    pid = pl.program_id(0)

    @pl.when(pid == 0)
    def _init():
        sums_ref[...] = jnp.zeros_like(sums_ref)
        counts_ref[...] = jnp.zeros_like(counts_ref)

    bt = batch_ref[0]
    oh = (lax.broadcasted_iota(jnp.int32, (nb, bn), 0) == bt).astype(jnp.float32)
    counts_ref[...] += jnp.dot(oh, jnp.ones((bn, 1), jnp.float32),
                               preferred_element_type=jnp.float32,
                               precision=lax.Precision.HIGHEST)
    dsel = (lax.broadcasted_iota(jnp.int32, (hh, hh * ch), 1) // ch ==
            lax.broadcasted_iota(jnp.int32, (hh, hh * ch), 0)).astype(jnp.float32)
    for c in range(NCORE):
        t2c = t2_ref[c]
        t48 = t2c[:, :hh * edim]
        denom = t2c[:, hh * edim:hh * edim + hh]
        agg = outv_ref[c] + jnp.dot(t48, wes_ref[c],
                                    preferred_element_type=jnp.float32,
                                    precision=lax.Precision.HIGHEST)
        dmat = jnp.dot(denom, dsel, preferred_element_type=jnp.float32,
                       precision=lax.Precision.HIGHEST) + 1e-16
        o = jnp.maximum(agg / dmat + skip_ref[c], 0.0)
        sums_ref[c, :, :] += jnp.dot(oh, o, preferred_element_type=jnp.float32,
                                     precision=lax.Precision.HIGHEST)


def _logits_body(sums_ref, counts_ref, wout_ref, bout_ref, o_ref):
    scale = 1.0 / jnp.maximum(counts_ref[...], 1.0)
    total = None
    for c in range(NCORE):
        pooled = sums_ref[c] * scale
        part = jnp.dot(pooled, wout_ref[c], preferred_element_type=jnp.float32,
                       precision=lax.Precision.HIGHEST)
        total = part if total is None else total + part
    o_ref[...] = total + bout_ref[0, 0]


def kernel(x, edge_attr, edge_weight, Wq, Wk, Wv, We, Wskip, Wout, bout,
           edge_index, batch):
    n, fin = x.shape
    e, edim = edge_attr.shape
    d = Wq.shape[1]
    ch = d // H
    nb = 64
    hh = H // NCORE
    half = hh * ch
    scale = 1.0 / np.sqrt(ch)

    be = 2048
    e_pad = ((e + be - 1) // be) * be
    per_worker = e_pad // NSUB
    n_chunks = per_worker // CHUNK
    rows_per_tile = ((n + NSUB - 1) // NSUB + 127) // 128 * 128
    n_pad = rows_per_tile * NSUB
    n_blk = e_pad // be

    src = edge_index[0]
    dst = edge_index[1]
    padlen = e_pad - e
    src_p = jnp.pad(src, (0, padlen)).astype(jnp.int32)
    dst_p = jnp.pad(dst, (0, padlen)).astype(jnp.int32)
    off = (jnp.arange(NCORE, dtype=jnp.int32) * n)[:, None]
    src2 = (src_p[None, :] + off).reshape(-1)
    dst2 = (dst_p[None, :] + off).reshape(-1)
    ea_p = jnp.pad(edge_attr, ((0, padlen), (0, 0)))
    ew_p = jnp.pad(edge_weight, (0, padlen))[:, None]
    wet = We.T
    wtmp = We.reshape(edim, NCORE, hh, ch).transpose(1, 2, 0, 3)
    wes = jnp.einsum("cldh,lk->cdlkh", wtmp, jnp.eye(hh, dtype=We.dtype))
    wes = wes.transpose(0, 2, 1, 3, 4).reshape(NCORE, hh * edim, half)
    wout2 = Wout.reshape(NCORE, half, 1)
    boutr = bout.reshape(1, 1)
    z128 = jnp.zeros((rows_per_tile, half), jnp.float32)
    batchr = batch.astype(jnp.int32).reshape(1, n)

    f32 = jnp.float32

    bn = 2000 if n % 2000 == 0 else n
    k1 = pl.pallas_call(
        functools.partial(_proj_body, scale=scale, ch=ch, edim=edim,
                          half=half),
        grid=(n // bn,),
        in_specs=[pl.BlockSpec((bn, fin), lambda i: (i, 0))] +
                 [pl.BlockSpec((fin, d), lambda i: (0, 0))] * 4 +
                 [pl.BlockSpec((d, edim), lambda i: (0, 0))],
        out_specs=[pl.BlockSpec((NCORE, bn, half), lambda i: (0, i, 0))] * 5,
        out_shape=[jax.ShapeDtypeStruct((NCORE, n, half), f32)] * 5,
    )
    q2, k2, v2, skip2, qwe2 = k1(x, Wq, Wk, Wv, Wskip, wet)

    mesh = plsc.VectorSubcoreMesh(core_axis_name="c", subcore_axis_name="s")
    gather = pl.kernel(
        functools.partial(_gather_body, n_chunks=n_chunks,
                          per_worker=per_worker, e_pad=e_pad),
        out_type=[jax.ShapeDtypeStruct((NCORE, e_pad, half), f32)] * 4,
        mesh=mesh,
        scratch_types=[
            pltpu.VMEM((CHUNK,), jnp.int32),
            pltpu.VMEM((CHUNK,), jnp.int32),
            pltpu.VMEM((CHUNK, half), f32),
            pltpu.VMEM((CHUNK, half), f32),
            pltpu.VMEM((CHUNK, half), f32),
            pltpu.VMEM((CHUNK, half), f32),
            pltpu.SemaphoreType.DMA,
            pltpu.SemaphoreType.DMA,
            pltpu.SemaphoreType.DMA,
            pltpu.SemaphoreType.DMA,
        ],
    )
    qd, ks, vs, qwed = gather(
        q2.reshape(NCORE * n, half), k2.reshape(NCORE * n, half),
        v2.reshape(NCORE * n, half), qwe2.reshape(NCORE * n, half),
        dst2, src2)

    k3 = pl.pallas_call(
        functools.partial(_alpha_body, be=be, n_edges=e, ch=ch, edim=edim),
        grid=(n_blk,),
        in_specs=[
            pl.BlockSpec((NCORE, be, half), lambda i: (0, i, 0)),
            pl.BlockSpec((NCORE, be, half), lambda i: (0, i, 0)),
            pl.BlockSpec((NCORE, be, half), lambda i: (0, i, 0)),
            pl.BlockSpec((be, edim), lambda i: (i, 0)),
        ],
        out_specs=[
            pl.BlockSpec((NCORE, be, hh), lambda i: (0, i, 0)),
            pl.BlockSpec((1, NCORE, hh), lambda i: (i, 0, 0)),
        ],
        out_shape=[jax.ShapeDtypeStruct((NCORE, e_pad, hh), f32),
                   jax.ShapeDtypeStruct((n_blk, NCORE, hh), f32)],
    )
    alpha, pm = k3(qd, ks, qwed, ea_p)

    k4 = pl.pallas_call(
        functools.partial(_payload_body, be=be, n_edges=e, ch=ch, edim=edim),
        grid=(n_blk,),
        in_specs=[
            pl.BlockSpec((NCORE, be, hh), lambda i: (0, i, 0)),
            pl.BlockSpec((n_blk, NCORE, hh), lambda i: (0, 0, 0)),
            pl.BlockSpec((NCORE, be, half), lambda i: (0, i, 0)),
            pl.BlockSpec((be, edim), lambda i: (i, 0)),
            pl.BlockSpec((be, 1), lambda i: (i, 0)),
        ],
        out_specs=[
            pl.BlockSpec((NCORE, be, half), lambda i: (0, i, 0)),
            pl.BlockSpec((NCORE, be, half), lambda i: (0, i, 0)),
        ],
        out_shape=[jax.ShapeDtypeStruct((NCORE, e_pad, half), f32),
                   jax.ShapeDtypeStruct((NCORE, e_pad, half), f32)],
    )
    msgv, pay2 = k4(alpha, pm, vs, ea_p, ew_p)

    def _mk_scatter(width):
        return pl.kernel(
            functools.partial(_scatter_body, n_chunks=n_chunks,
                              per_worker=per_worker,
                              rows_per_tile=rows_per_tile),
            out_type=jax.ShapeDtypeStruct((NCORE, n_pad, width), f32),
            mesh=mesh,
            scratch_types=[
                pltpu.VMEM_SHARED((n_pad, width), f32),
                pltpu.VMEM((CHUNK,), jnp.int32),
                pltpu.VMEM((CHUNK, width), f32),
            ],
        )
    sc5 = _mk_scatter(half)
    outv = sc5(msgv, dst_p, z128)
    t2 = sc5(pay2, dst_p, z128)

    bn = 2000 if n % 2000 == 0 else n
    k6a = pl.pallas_call(
        functools.partial(_combine_body, bn=bn, nb=nb, ch=ch, edim=edim),
        grid=(n // bn,),
        in_specs=[
            pl.BlockSpec((NCORE, bn, half), lambda i: (0, i, 0)),
            pl.BlockSpec((NCORE, bn, half), lambda i: (0, i, 0)),
            pl.BlockSpec((NCORE, bn, half), lambda i: (0, i, 0)),
            pl.BlockSpec((NCORE, hh * edim, half), lambda i: (0, 0, 0)),
            pl.BlockSpec((1, 1, bn), lambda i: (i, 0, 0)),
        ],
        out_specs=[
            pl.BlockSpec((NCORE, nb, half), lambda i: (0, 0, 0)),
            pl.BlockSpec((nb, 1), lambda i: (0, 0)),
        ],
        out_shape=[jax.ShapeDtypeStruct((NCORE, nb, half), f32),
                   jax.ShapeDtypeStruct((nb, 1), f32)],
    )
    sums, counts = k6a(outv[:, :n], t2[:, :n], skip2, wes,
                       batchr.reshape(n // bn, 1, bn))

    k6b = pl.pallas_call(
        _logits_body,
        out_shape=jax.ShapeDtypeStruct((nb, 1), f32),
    )
    logits = k6b(sums, counts, wout2, boutr)
    return logits[:, 0]
